# baseline (speedup 1.0000x reference)
import jax
import jax.numpy as jnp
from jax import lax
from jax.experimental import pallas as pl
from jax.experimental.pallas import tpu as pltpu
from jax.experimental.pallas import tpu_sc as plsc

N = 512
E = 16384

N_SUB = 16
EDGES_PER_TILE = E // N_SUB
SCAT_BATCH = 128
N_SCAT = EDGES_PER_TILE // SCAT_BATCH
C_SLICE = (N * N) // N_SUB

K1_BLK = 256
N_K1 = 16384 // K1_BLK
K2_BLK = 1024
N_K2 = 4096 // K2_BLK
N_STEPS = N_K1 + N_K2


def _adj_sc(src_hbm, dst_hbm, zeros_hbm, c_hbm, sidx, didx, fidx, ones_v,
            shared_c):
    core = lax.axis_index("c")
    s = lax.axis_index("s")

    @pl.when(core == 0)
    def _():
        base = s * EDGES_PER_TILE

        pltpu.sync_copy(zeros_hbm.at[pl.ds(s * C_SLICE, C_SLICE)],
                        shared_c.at[pl.ds(s * C_SLICE, C_SLICE)])

        pltpu.sync_copy(src_hbm.at[pl.ds(base, EDGES_PER_TILE)], sidx)
        pltpu.sync_copy(dst_hbm.at[pl.ds(base, EDGES_PER_TILE)], didx)

        for i in range(SCAT_BATCH // 16):
            ones_v[pl.ds(i * 16, 16)] = jnp.full((16,), 1.0, jnp.float32)
        for j in range(N_SCAT):
            for i in range(SCAT_BATCH // 16):
                off = j * SCAT_BATCH + i * 16
                fidx[j, pl.ds(i * 16, 16)] = (
                    didx[pl.ds(off, 16)] * N + sidx[pl.ds(off, 16)])

        plsc.subcore_barrier()
        for j in range(N_SCAT):
            pltpu.sync_copy(ones_v, shared_c.at[fidx.at[j]], add=True)
        plsc.subcore_barrier()

        pltpu.sync_copy(shared_c.at[pl.ds(s * C_SLICE, C_SLICE)],
                        c_hbm.at[pl.ds(s * C_SLICE, C_SLICE)])


def _fused_kernel(c_ref, x_ref, win_ref, bin_ref, wb_ref, bb_ref,
                  w1_ref, b1_ref, w2_ref, b2_ref, w3_ref, b3_ref,
                  o_ref, v_s, o1_s, o2_s):
    k = pl.program_id(0)

    @pl.when(k == 0)
    def _():
        eye = (jax.lax.broadcasted_iota(jnp.int32, (N, N), 0)
               == jax.lax.broadcasted_iota(jnp.int32, (N, N), 1))
        C = c_ref[...] + eye.astype(jnp.float32)
        deg = jnp.sum(C, axis=1)
        dinv = jax.lax.rsqrt(jnp.maximum(deg, 1e-12))
        A = C * dinv[:, None] * dinv[None, :]

        def conv(h, W, b):
            hw = jnp.dot(h, W, preferred_element_type=jnp.float32)
            return jnp.dot(A, hw, preferred_element_type=jnp.float32) + b

        h = jnp.maximum(conv(x_ref[...], win_ref[...], bin_ref[0, :]), 0.0)
        for i in range(3):
            t = jnp.maximum(conv(h, wb_ref[2 * i], bb_ref[2 * i]), 0.0)
            t = conv(t, wb_ref[2 * i + 1], bb_ref[2 * i + 1])
            h = jnp.maximum(t + h, 0.0)
        v_s[...] = h
        o1_s[...] = jnp.zeros_like(o1_s)
        o2_s[...] = jnp.zeros_like(o2_s)

    @pl.when(k < N_K1)
    def _():
        nsub = K1_BLK // 32
        parts = [
            jnp.dot(v_s[pl.ds(nsub * k + j, 1), :],
                    w1_ref[pl.ds(32 * j, 32), :],
                    preferred_element_type=jnp.float32)
            for j in range(nsub)
        ]
        while len(parts) > 1:
            parts = [a + b for a, b in zip(parts[::2], parts[1::2])]
        o1_s[...] += parts[0]

    @pl.when(k == N_K1 - 1)
    def _():
        o1_s[...] = jnp.maximum(o1_s[...] + b1_ref[...], 0.0)

    @pl.when(k >= N_K1)
    def _():
        j = k - N_K1
        vblk = o1_s[:, pl.ds(pl.multiple_of(j * K2_BLK, K2_BLK), K2_BLK)]
        o2_s[...] += jnp.dot(vblk, w2_ref[...],
                             preferred_element_type=jnp.float32)

    @pl.when(k == N_STEPS - 1)
    def _():
        v2 = jnp.maximum(o2_s[...] + b2_ref[...], 0.0)
        logits = (jnp.dot(v2, w3_ref[...],
                          preferred_element_type=jnp.float32) + b3_ref[...])
        m = jnp.max(logits, axis=-1, keepdims=True)
        e = jnp.exp(logits - m)
        o_ref[...] = e / jnp.sum(e, axis=-1, keepdims=True)


def kernel(x, edge_index, W_in, b_in, Wb, bb, W1, b1, W2, b2, W3, b3):
    src = edge_index[0]
    dst = edge_index[1]
    zeros = jnp.zeros((N * N,), jnp.float32)

    c_flat = pl.kernel(
        _adj_sc,
        out_type=jax.ShapeDtypeStruct((N * N,), jnp.float32),
        mesh=plsc.VectorSubcoreMesh(core_axis_name="c",
                                    subcore_axis_name="s"),
        scratch_types=[
            pltpu.VMEM((EDGES_PER_TILE,), jnp.int32),
            pltpu.VMEM((EDGES_PER_TILE,), jnp.int32),
            pltpu.VMEM((N_SCAT, SCAT_BATCH), jnp.int32),
            pltpu.VMEM((SCAT_BATCH,), jnp.float32),
            pltpu.VMEM_SHARED((N * N,), jnp.float32),
        ],
    )(src, dst, zeros)

    C = c_flat.reshape(N, N)

    out = pl.pallas_call(
        _fused_kernel,
        grid=(N_STEPS,),
        in_specs=[
            pl.BlockSpec((N, N), lambda k: (0, 0)),
            pl.BlockSpec((N, 64), lambda k: (0, 0)),
            pl.BlockSpec((64, 32), lambda k: (0, 0)),
            pl.BlockSpec((1, 32), lambda k: (0, 0)),
            pl.BlockSpec((6, 32, 32), lambda k: (0, 0, 0)),
            pl.BlockSpec((6, 32), lambda k: (0, 0)),
            pl.BlockSpec((K1_BLK, 4096),
                         lambda k: (jnp.minimum(k, N_K1 - 1), 0)),
            pl.BlockSpec((1, 4096), lambda k: (0, 0)),
            pl.BlockSpec((K2_BLK, 2048),
                         lambda k: (jnp.clip(k - N_K1, 0, N_K2 - 1), 0)),
            pl.BlockSpec((1, 2048), lambda k: (0, 0)),
            pl.BlockSpec((2048, 10), lambda k: (0, 0)),
            pl.BlockSpec((1, 10), lambda k: (0, 0)),
        ],
        out_specs=pl.BlockSpec((1, 10), lambda k: (0, 0)),
        out_shape=jax.ShapeDtypeStruct((1, 10), jnp.float32),
        scratch_shapes=[
            pltpu.VMEM((N, 32), jnp.float32),
            pltpu.VMEM((1, 4096), jnp.float32),
            pltpu.VMEM((1, 2048), jnp.float32),
        ],
    )(C, x, W_in, b_in.reshape(1, 32), Wb.reshape(6, 32, 32),
      bb.reshape(6, 32), W1, b1.reshape(1, -1), W2, b2.reshape(1, -1),
      W3, b3.reshape(1, -1))

    return out.reshape(10)


# speedup vs baseline: 1.0874x; 1.0874x over previous
import jax
import jax.numpy as jnp
from jax import lax
from jax.experimental import pallas as pl
from jax.experimental.pallas import tpu as pltpu
from jax.experimental.pallas import tpu_sc as plsc

N = 512
E = 16384

N_SUB = 16
EDGES_PER_TILE = E // N_SUB
SCAT_BATCH = 128
N_SCAT = EDGES_PER_TILE // SCAT_BATCH
C_SLICE = (N * N) // N_SUB

K1_BLK = 512
N_K1 = 16384 // K1_BLK
K2_BLK = 1024
N_K2 = 4096 // K2_BLK
N_STEPS = N_K1 + N_K2


def _adj_sc(src_hbm, dst_hbm, zeros_hbm, c_hbm, sidx, didx, fidx, ones_v,
            shared_c, in_sem, scat_sem):
    core = lax.axis_index("c")
    s = lax.axis_index("s")

    @pl.when(core == 0)
    def _():
        base = s * EDGES_PER_TILE

        cz = pltpu.async_copy(zeros_hbm.at[pl.ds(s * C_SLICE, C_SLICE)],
                              shared_c.at[pl.ds(s * C_SLICE, C_SLICE)],
                              in_sem)
        cs = pltpu.async_copy(src_hbm.at[pl.ds(base, EDGES_PER_TILE)],
                              sidx, in_sem)
        cd = pltpu.async_copy(dst_hbm.at[pl.ds(base, EDGES_PER_TILE)],
                              didx, in_sem)

        for i in range(SCAT_BATCH // 16):
            ones_v[pl.ds(i * 16, 16)] = jnp.full((16,), 1.0, jnp.float32)

        cs.wait()
        cd.wait()
        for j in range(N_SCAT):
            for i in range(SCAT_BATCH // 16):
                off = j * SCAT_BATCH + i * 16
                fidx[j, pl.ds(i * 16, 16)] = (
                    didx[pl.ds(off, 16)] * N + sidx[pl.ds(off, 16)])
        cz.wait()

        plsc.subcore_barrier()
        scats = [pltpu.async_copy(ones_v, shared_c.at[fidx.at[j]],
                                  scat_sem, add=True)
                 for j in range(N_SCAT)]
        for c in scats:
            c.wait()
        plsc.subcore_barrier()

        pltpu.sync_copy(shared_c.at[pl.ds(s * C_SLICE, C_SLICE)],
                        c_hbm.at[pl.ds(s * C_SLICE, C_SLICE)])


def _fused_kernel(c_ref, x_ref, win_ref, bin_ref, wb_ref, bb_ref,
                  w1_ref, b1_ref, w2_ref, b2_ref, w3_ref, b3_ref,
                  o_ref, v_s, o1_s, o2_s):
    k = pl.program_id(0)

    @pl.when(k == 0)
    def _():
        eye = (jax.lax.broadcasted_iota(jnp.int32, (N, N), 0)
               == jax.lax.broadcasted_iota(jnp.int32, (N, N), 1))
        C = c_ref[...] + eye.astype(jnp.float32)
        deg = jnp.sum(C, axis=1)
        dinv = jax.lax.rsqrt(jnp.maximum(deg, 1e-12))
        A = C * dinv[:, None] * dinv[None, :]

        def conv(h, W, b):
            hw = jnp.dot(h, W, preferred_element_type=jnp.float32)
            return jnp.dot(A, hw, preferred_element_type=jnp.float32) + b

        h = jnp.maximum(conv(x_ref[...], win_ref[...], bin_ref[0, :]), 0.0)
        for i in range(3):
            t = jnp.maximum(conv(h, wb_ref[2 * i], bb_ref[2 * i]), 0.0)
            t = conv(t, wb_ref[2 * i + 1], bb_ref[2 * i + 1])
            h = jnp.maximum(t + h, 0.0)
        v_s[...] = h
        o1_s[...] = jnp.zeros_like(o1_s)
        o2_s[...] = jnp.zeros_like(o2_s)

    @pl.when(k < N_K1)
    def _():
        nsub = K1_BLK // 32
        parts = [
            jnp.dot(v_s[pl.ds(nsub * k + j, 1), :],
                    w1_ref[pl.ds(32 * j, 32), :],
                    preferred_element_type=jnp.float32)
            for j in range(nsub)
        ]
        while len(parts) > 1:
            parts = [a + b for a, b in zip(parts[::2], parts[1::2])]
        o1_s[...] += parts[0]

    @pl.when(k == N_K1 - 1)
    def _():
        o1_s[...] = jnp.maximum(o1_s[...] + b1_ref[...], 0.0)

    @pl.when(k >= N_K1)
    def _():
        j = k - N_K1
        vblk = o1_s[:, pl.ds(pl.multiple_of(j * K2_BLK, K2_BLK), K2_BLK)]
        o2_s[...] += jnp.dot(vblk, w2_ref[...],
                             preferred_element_type=jnp.float32)

    @pl.when(k == N_STEPS - 1)
    def _():
        v2 = jnp.maximum(o2_s[...] + b2_ref[...], 0.0)
        logits = (jnp.dot(v2, w3_ref[...],
                          preferred_element_type=jnp.float32) + b3_ref[...])
        m = jnp.max(logits, axis=-1, keepdims=True)
        e = jnp.exp(logits - m)
        o_ref[...] = e / jnp.sum(e, axis=-1, keepdims=True)


def kernel(x, edge_index, W_in, b_in, Wb, bb, W1, b1, W2, b2, W3, b3):
    src = edge_index[0]
    dst = edge_index[1]
    zeros = jnp.zeros((N * N,), jnp.float32)

    c_flat = pl.kernel(
        _adj_sc,
        out_type=jax.ShapeDtypeStruct((N * N,), jnp.float32),
        mesh=plsc.VectorSubcoreMesh(core_axis_name="c",
                                    subcore_axis_name="s"),
        scratch_types=[
            pltpu.VMEM((EDGES_PER_TILE,), jnp.int32),
            pltpu.VMEM((EDGES_PER_TILE,), jnp.int32),
            pltpu.VMEM((N_SCAT, SCAT_BATCH), jnp.int32),
            pltpu.VMEM((SCAT_BATCH,), jnp.float32),
            pltpu.VMEM_SHARED((N * N,), jnp.float32),
            pltpu.SemaphoreType.DMA,
            pltpu.SemaphoreType.DMA,
        ],
    )(src, dst, zeros)

    C = c_flat.reshape(N, N)

    out = pl.pallas_call(
        _fused_kernel,
        grid=(N_STEPS,),
        in_specs=[
            pl.BlockSpec((N, N), lambda k: (0, 0)),
            pl.BlockSpec((N, 64), lambda k: (0, 0)),
            pl.BlockSpec((64, 32), lambda k: (0, 0)),
            pl.BlockSpec((1, 32), lambda k: (0, 0)),
            pl.BlockSpec((6, 32, 32), lambda k: (0, 0, 0)),
            pl.BlockSpec((6, 32), lambda k: (0, 0)),
            pl.BlockSpec((K1_BLK, 4096),
                         lambda k: (jnp.minimum(k, N_K1 - 1), 0)),
            pl.BlockSpec((1, 4096), lambda k: (0, 0)),
            pl.BlockSpec((K2_BLK, 2048),
                         lambda k: (jnp.clip(k - N_K1, 0, N_K2 - 1), 0)),
            pl.BlockSpec((1, 2048), lambda k: (0, 0)),
            pl.BlockSpec((2048, 10), lambda k: (0, 0)),
            pl.BlockSpec((1, 10), lambda k: (0, 0)),
        ],
        out_specs=pl.BlockSpec((1, 10), lambda k: (0, 0)),
        out_shape=jax.ShapeDtypeStruct((1, 10), jnp.float32),
        scratch_shapes=[
            pltpu.VMEM((N, 32), jnp.float32),
            pltpu.VMEM((1, 4096), jnp.float32),
            pltpu.VMEM((1, 2048), jnp.float32),
        ],
    )(C, x, W_in, b_in.reshape(1, 32), Wb.reshape(6, 32, 32),
      bb.reshape(6, 32), W1, b1.reshape(1, -1), W2, b2.reshape(1, -1),
      W3, b3.reshape(1, -1))

    return out.reshape(10)


# speedup vs baseline: 1.0903x; 1.0026x over previous
import jax
import jax.numpy as jnp
from jax import lax
from jax.experimental import pallas as pl
from jax.experimental.pallas import tpu as pltpu
from jax.experimental.pallas import tpu_sc as plsc

N = 512
E = 16384

N_SUB = 16
EDGES_PER_TILE = E // N_SUB
SCAT_BATCH = 128
N_SCAT = EDGES_PER_TILE // SCAT_BATCH
C_SLICE = (N * N) // N_SUB

K1_BLK = 512
N_K1 = 16384 // K1_BLK
K2_BLK = 1024
N_K2 = 4096 // K2_BLK
N_STEPS = N_K1 + N_K2


def _adj_sc(src_hbm, dst_hbm, zeros_hbm, c_hbm, sidx, didx, fidx, ones_v,
            shared_c, in_sem, scat_sem):
    core = lax.axis_index("c")
    s = lax.axis_index("s")

    @pl.when(core == 0)
    def _():
        base = s * EDGES_PER_TILE

        cz = pltpu.async_copy(zeros_hbm,
                              shared_c.at[pl.ds(s * C_SLICE, C_SLICE)],
                              in_sem)
        cs = pltpu.async_copy(src_hbm.at[pl.ds(base, EDGES_PER_TILE)],
                              sidx, in_sem)
        cd = pltpu.async_copy(dst_hbm.at[pl.ds(base, EDGES_PER_TILE)],
                              didx, in_sem)

        for i in range(SCAT_BATCH // 16):
            ones_v[pl.ds(i * 16, 16)] = jnp.full((16,), 1.0, jnp.float32)

        cs.wait()
        cd.wait()
        for j in range(N_SCAT):
            for i in range(SCAT_BATCH // 16):
                off = j * SCAT_BATCH + i * 16
                fidx[j, pl.ds(i * 16, 16)] = (
                    didx[pl.ds(off, 16)] * N + sidx[pl.ds(off, 16)])
        cz.wait()

        plsc.subcore_barrier()
        scats = [pltpu.async_copy(ones_v, shared_c.at[fidx.at[j]],
                                  scat_sem, add=True)
                 for j in range(N_SCAT)]
        for c in scats:
            c.wait()
        plsc.subcore_barrier()

        pltpu.sync_copy(shared_c.at[pl.ds(s * C_SLICE, C_SLICE)],
                        c_hbm.at[pl.ds(s * C_SLICE, C_SLICE)])


def _fused_kernel(c_ref, x_ref, win_ref, bin_ref, wb_ref, bb_ref,
                  w1_ref, b1_ref, w2_ref, b2_ref, w3_ref, b3_ref,
                  o_ref, v_s, o1_s, o2_s):
    k = pl.program_id(0)

    @pl.when(k == 0)
    def _():
        eye = (jax.lax.broadcasted_iota(jnp.int32, (N, N), 0)
               == jax.lax.broadcasted_iota(jnp.int32, (N, N), 1))
        C = c_ref[...] + eye.astype(jnp.float32)
        deg = jnp.sum(C, axis=1)
        dinv = jax.lax.rsqrt(jnp.maximum(deg, 1e-12))
        A = C * dinv[:, None] * dinv[None, :]

        def conv(h, W, b):
            hw = jnp.dot(h, W, preferred_element_type=jnp.float32)
            return jnp.dot(A, hw, preferred_element_type=jnp.float32) + b

        h = jnp.maximum(conv(x_ref[...], win_ref[...], bin_ref[0, :]), 0.0)
        for i in range(3):
            t = jnp.maximum(conv(h, wb_ref[2 * i], bb_ref[2 * i]), 0.0)
            t = conv(t, wb_ref[2 * i + 1], bb_ref[2 * i + 1])
            h = jnp.maximum(t + h, 0.0)
        v_s[...] = h
        o1_s[...] = jnp.zeros_like(o1_s)
        o2_s[...] = jnp.zeros_like(o2_s)

    @pl.when(k < N_K1)
    def _():
        nsub = K1_BLK // 32
        parts = [
            jnp.dot(v_s[pl.ds(nsub * k + j, 1), :],
                    w1_ref[pl.ds(32 * j, 32), :],
                    preferred_element_type=jnp.float32)
            for j in range(nsub)
        ]
        while len(parts) > 1:
            parts = [a + b for a, b in zip(parts[::2], parts[1::2])]
        o1_s[...] += parts[0]

    @pl.when(k == N_K1 - 1)
    def _():
        o1_s[...] = jnp.maximum(o1_s[...] + b1_ref[...], 0.0)

    @pl.when(k >= N_K1)
    def _():
        j = k - N_K1
        vblk = o1_s[:, pl.ds(pl.multiple_of(j * K2_BLK, K2_BLK), K2_BLK)]
        o2_s[...] += jnp.dot(vblk, w2_ref[...],
                             preferred_element_type=jnp.float32)

    @pl.when(k == N_STEPS - 1)
    def _():
        v2 = jnp.maximum(o2_s[...] + b2_ref[...], 0.0)
        logits = (jnp.dot(v2, w3_ref[...],
                          preferred_element_type=jnp.float32) + b3_ref[...])
        m = jnp.max(logits, axis=-1, keepdims=True)
        e = jnp.exp(logits - m)
        o_ref[...] = e / jnp.sum(e, axis=-1, keepdims=True)


def kernel(x, edge_index, W_in, b_in, Wb, bb, W1, b1, W2, b2, W3, b3):
    src = edge_index[0]
    dst = edge_index[1]
    zeros = jnp.zeros((C_SLICE,), jnp.float32)

    c_flat = pl.kernel(
        _adj_sc,
        out_type=jax.ShapeDtypeStruct((N * N,), jnp.float32),
        mesh=plsc.VectorSubcoreMesh(core_axis_name="c",
                                    subcore_axis_name="s"),
        scratch_types=[
            pltpu.VMEM((EDGES_PER_TILE,), jnp.int32),
            pltpu.VMEM((EDGES_PER_TILE,), jnp.int32),
            pltpu.VMEM((N_SCAT, SCAT_BATCH), jnp.int32),
            pltpu.VMEM((SCAT_BATCH,), jnp.float32),
            pltpu.VMEM_SHARED((N * N,), jnp.float32),
            pltpu.SemaphoreType.DMA,
            pltpu.SemaphoreType.DMA,
        ],
    )(src, dst, zeros)

    C = c_flat.reshape(N, N)

    out = pl.pallas_call(
        _fused_kernel,
        grid=(N_STEPS,),
        in_specs=[
            pl.BlockSpec((N, N), lambda k: (0, 0)),
            pl.BlockSpec((N, 64), lambda k: (0, 0)),
            pl.BlockSpec((64, 32), lambda k: (0, 0)),
            pl.BlockSpec((1, 32), lambda k: (0, 0)),
            pl.BlockSpec((6, 32, 32), lambda k: (0, 0, 0)),
            pl.BlockSpec((6, 32), lambda k: (0, 0)),
            pl.BlockSpec((K1_BLK, 4096),
                         lambda k: (jnp.minimum(k, N_K1 - 1), 0)),
            pl.BlockSpec((1, 4096), lambda k: (0, 0)),
            pl.BlockSpec((K2_BLK, 2048),
                         lambda k: (jnp.clip(k - N_K1, 0, N_K2 - 1), 0)),
            pl.BlockSpec((1, 2048), lambda k: (0, 0)),
            pl.BlockSpec((2048, 10), lambda k: (0, 0)),
            pl.BlockSpec((1, 10), lambda k: (0, 0)),
        ],
        out_specs=pl.BlockSpec((1, 10), lambda k: (0, 0)),
        out_shape=jax.ShapeDtypeStruct((1, 10), jnp.float32),
        scratch_shapes=[
            pltpu.VMEM((N, 32), jnp.float32),
            pltpu.VMEM((1, 4096), jnp.float32),
            pltpu.VMEM((1, 2048), jnp.float32),
        ],
    )(C, x, W_in, b_in.reshape(1, 32), Wb.reshape(6, 32, 32),
      bb.reshape(6, 32), W1, b1.reshape(1, -1), W2, b2.reshape(1, -1),
      W3, b3.reshape(1, -1))

    return out.reshape(10)
